# baseline (speedup 1.0000x reference)
import functools

import jax
import jax.numpy as jnp
from jax import lax
from jax.experimental import pallas as pl
from jax.experimental.pallas import tpu as pltpu
from jax.experimental.pallas import tpu_sc as plsc

NC = 2
NS = 16
NW = NC * NS
CHUNK = 128
LBLK = 8192


def _reformat_body(tin_ref, z_ref):
    s = jnp.concatenate([tin_ref[:, 0:4096], tin_ref[:, 4096:8192]], axis=0)
    z_ref[...] = s.T.astype(jnp.bfloat16)


def _tc_reformat(tbl_t):
    D, V = tbl_t.shape
    grid = (V + LBLK - 1) // LBLK
    return pl.pallas_call(
        _reformat_body,
        grid=(grid,),
        in_specs=[pl.BlockSpec((D, LBLK), lambda j: (0, j))],
        out_specs=pl.BlockSpec((LBLK // 2, 128), lambda j: (j, 0)),
        out_shape=jax.ShapeDtypeStruct((grid * (LBLK // 2), 128), jnp.bfloat16),
    )(tbl_t)


def kernel(condition, learned_embs, uncond_embedding):
    B = condition.shape[0]
    V, D = learned_embs.shape
    b_per_w = B // NW
    n_chunks = b_per_w // CHUNK

    idx = condition.astype(jnp.int32).reshape(NW, n_chunks, CHUNK)
    z = _tc_reformat(learned_embs.T)
    z64 = z.reshape(z.shape[0] * 2, D)

    mesh = plsc.VectorSubcoreMesh(core_axis_name="c", subcore_axis_name="s")

    @functools.partial(
        pl.kernel,
        mesh=mesh,
        out_type=jax.ShapeDtypeStruct((B, D), jnp.bfloat16),
        scratch_types=[
            pltpu.VMEM((n_chunks, CHUNK), jnp.int32),
            pltpu.VMEM((n_chunks, CHUNK), jnp.int32),
            pltpu.VMEM((b_per_w, D), jnp.bfloat16),
            pltpu.SemaphoreType.DMA,
        ],
        compiler_params=pltpu.CompilerParams(use_tc_tiling_on_sc=False),
    )
    def emb_gather(idx_hbm, table_hbm, out_hbm, idx_v, row_v, rows_v, sem):
        wid = lax.axis_index("s") * NC + lax.axis_index("c")
        pltpu.sync_copy(idx_hbm.at[wid], idx_v)
        for j in range(n_chunks):
            for k in range(CHUNK // 16):
                r = idx_v[j, pl.ds(k * 16, 16)]
                rr = ((r >> 13) << 13) + ((r & 4095) << 1) + ((r >> 12) & 1)
                row_v[j, pl.ds(k * 16, 16)] = rr
        copies = [
            pltpu.async_copy(
                table_hbm.at[row_v.at[j]],
                rows_v.at[pl.ds(j * CHUNK, CHUNK)],
                sem,
            )
            for j in range(n_chunks)
        ]
        for cp in copies:
            cp.wait()
        pltpu.sync_copy(rows_v, out_hbm.at[pl.ds(wid * b_per_w, b_per_w)])

    return emb_gather(idx, z64).astype(jnp.float32)


# speedup vs baseline: 3.0077x; 3.0077x over previous
import functools

import jax
import jax.numpy as jnp
from jax import lax
from jax.experimental import pallas as pl
from jax.experimental.pallas import tpu as pltpu
from jax.experimental.pallas import tpu_sc as plsc

NC = 2
NS = 16
NW = NC * NS
CHUNK = 128
LBLK = 16384
HALF = LBLK // 2


def _reformat_body(tin_ref, z_ref):
    s = jnp.concatenate([tin_ref[:, 0:HALF], tin_ref[:, HALF:LBLK]], axis=0)
    z_ref[...] = s.T


def _tc_reformat(tbl_t):
    D, V = tbl_t.shape
    grid = (V + LBLK - 1) // LBLK
    return pl.pallas_call(
        _reformat_body,
        grid=(grid,),
        in_specs=[pl.BlockSpec((D, LBLK), lambda j: (0, j))],
        out_specs=pl.BlockSpec((HALF, 128), lambda j: (j, 0)),
        out_shape=jax.ShapeDtypeStruct((grid * HALF, 128), jnp.float32),
    )(tbl_t)


def kernel(condition, learned_embs, uncond_embedding):
    B = condition.shape[0]
    V, D = learned_embs.shape
    b_per_w = B // NW
    n_chunks = b_per_w // CHUNK

    idx = condition.astype(jnp.int32).reshape(NW, n_chunks, CHUNK)
    z = _tc_reformat(learned_embs.T)
    z64 = z.reshape(z.shape[0] * 2, D)

    mesh = plsc.VectorSubcoreMesh(core_axis_name="c", subcore_axis_name="s")

    @functools.partial(
        pl.kernel,
        mesh=mesh,
        out_type=jax.ShapeDtypeStruct((B, D), jnp.float32),
        scratch_types=[
            pltpu.VMEM((n_chunks, CHUNK), jnp.int32),
            pltpu.VMEM((n_chunks, CHUNK), jnp.int32),
            pltpu.VMEM((b_per_w, D), jnp.float32),
            pltpu.SemaphoreType.DMA,
        ],
        compiler_params=pltpu.CompilerParams(use_tc_tiling_on_sc=False),
    )
    def emb_gather(idx_hbm, table_hbm, out_hbm, idx_v, row_v, rows_v, sem):
        wid = lax.axis_index("s") * NC + lax.axis_index("c")
        pltpu.sync_copy(idx_hbm.at[wid], idx_v)
        for j in range(n_chunks):
            for k in range(CHUNK // 16):
                r = idx_v[j, pl.ds(k * 16, 16)]
                rr = ((r >> 14) << 14) + ((r & 8191) << 1) + ((r >> 13) & 1)
                row_v[j, pl.ds(k * 16, 16)] = rr
        copies = [
            pltpu.async_copy(
                table_hbm.at[row_v.at[j]],
                rows_v.at[pl.ds(j * CHUNK, CHUNK)],
                sem,
            )
            for j in range(n_chunks)
        ]
        for cp in copies:
            cp.wait()
        pltpu.sync_copy(rows_v, out_hbm.at[pl.ds(wid * b_per_w, b_per_w)])

    return emb_gather(idx, z64)


# speedup vs baseline: 3.0861x; 1.0260x over previous
import functools

import jax
import jax.numpy as jnp
from jax import lax
from jax.experimental import pallas as pl
from jax.experimental.pallas import tpu as pltpu
from jax.experimental.pallas import tpu_sc as plsc

NC = 2
NS = 16
NW = NC * NS
CHUNK = 128
LBLK = 32768
HALF = LBLK // 2


def _reformat_body(tin_ref, z_ref):
    s = jnp.concatenate([tin_ref[:, 0:HALF], tin_ref[:, HALF:LBLK]], axis=0)
    z_ref[...] = s.T


def _tc_reformat(tbl_t):
    D, V = tbl_t.shape
    grid = (V + LBLK - 1) // LBLK
    return pl.pallas_call(
        _reformat_body,
        grid=(grid,),
        in_specs=[pl.BlockSpec((D, LBLK), lambda j: (0, j))],
        out_specs=pl.BlockSpec((HALF, 128), lambda j: (j, 0)),
        out_shape=jax.ShapeDtypeStruct((grid * HALF, 128), jnp.float32),
    )(tbl_t)


def kernel(condition, learned_embs, uncond_embedding):
    B = condition.shape[0]
    V, D = learned_embs.shape
    b_per_w = B // NW
    n_chunks = b_per_w // CHUNK

    idx = condition.astype(jnp.int32).reshape(NW, n_chunks, CHUNK)
    z = _tc_reformat(learned_embs.T)
    z64 = z.reshape(z.shape[0] * 2, D)

    mesh = plsc.VectorSubcoreMesh(core_axis_name="c", subcore_axis_name="s")

    @functools.partial(
        pl.kernel,
        mesh=mesh,
        out_type=jax.ShapeDtypeStruct((B, D), jnp.float32),
        scratch_types=[
            pltpu.VMEM((n_chunks, CHUNK), jnp.int32),
            pltpu.VMEM((n_chunks, CHUNK), jnp.int32),
            pltpu.VMEM((b_per_w, D), jnp.float32),
            pltpu.SemaphoreType.DMA,
        ],
        compiler_params=pltpu.CompilerParams(use_tc_tiling_on_sc=False),
    )
    def emb_gather(idx_hbm, table_hbm, out_hbm, idx_v, row_v, rows_v, sem):
        wid = lax.axis_index("s") * NC + lax.axis_index("c")
        pltpu.sync_copy(idx_hbm.at[wid], idx_v)
        for j in range(n_chunks):
            for k in range(CHUNK // 16):
                r = idx_v[j, pl.ds(k * 16, 16)]
                rr = ((r >> 15) << 15) + ((r & 16383) << 1) + ((r >> 14) & 1)
                row_v[j, pl.ds(k * 16, 16)] = rr
        copies = [
            pltpu.async_copy(
                table_hbm.at[row_v.at[j]],
                rows_v.at[pl.ds(j * CHUNK, CHUNK)],
                sem,
            )
            for j in range(n_chunks)
        ]
        for cp in copies:
            cp.wait()
        pltpu.sync_copy(rows_v, out_hbm.at[pl.ds(wid * b_per_w, b_per_w)])

    return emb_gather(idx, z64)


# speedup vs baseline: 3.1389x; 1.0171x over previous
import functools

import jax
import jax.numpy as jnp
from jax import lax
from jax.experimental import pallas as pl
from jax.experimental.pallas import tpu as pltpu
from jax.experimental.pallas import tpu_sc as plsc

NC = 2
NS = 16
NW = NC * NS
CHUNK = 128
LBLK = 32768
HALF = LBLK // 2


def _reformat_body(tin_ref, z_ref):
    s = jnp.concatenate([tin_ref[:, 0:HALF], tin_ref[:, HALF:LBLK]], axis=0)
    zb = s.T.astype(jnp.bfloat16)
    z_ref[...] = pltpu.bitcast(zb, jnp.float32)


def _tc_reformat(tbl_t):
    D, V = tbl_t.shape
    grid = (V + LBLK - 1) // LBLK
    return pl.pallas_call(
        _reformat_body,
        grid=(grid,),
        in_specs=[pl.BlockSpec((D, LBLK), lambda j: (0, j))],
        out_specs=pl.BlockSpec((HALF // 2, 128), lambda j: (j, 0)),
        out_shape=jax.ShapeDtypeStruct((grid * (HALF // 2), 128), jnp.float32),
    )(tbl_t)


def kernel(condition, learned_embs, uncond_embedding):
    B = condition.shape[0]
    V, D = learned_embs.shape
    b_per_w = B // NW
    n_chunks = b_per_w // CHUNK

    idx = condition.astype(jnp.int32).reshape(NW, n_chunks, CHUNK)
    z = _tc_reformat(learned_embs.T)
    z64 = z.reshape(z.shape[0] * 2, D)

    mesh = plsc.VectorSubcoreMesh(core_axis_name="c", subcore_axis_name="s")

    @functools.partial(
        pl.kernel,
        mesh=mesh,
        out_type=jax.ShapeDtypeStruct((B, D), jnp.float32),
        scratch_types=[
            pltpu.VMEM((n_chunks, CHUNK), jnp.int32),
            pltpu.VMEM((n_chunks, CHUNK), jnp.int32),
            pltpu.VMEM((n_chunks, CHUNK), jnp.int32),
            pltpu.VMEM((b_per_w, D), jnp.float32),
            pltpu.VMEM((b_per_w, D), jnp.float32),
            pltpu.SemaphoreType.DMA,
        ],
        compiler_params=pltpu.CompilerParams(
            use_tc_tiling_on_sc=False, needs_layout_passes=False
        ),
    )
    def emb_gather(idx_hbm, table_hbm, out_hbm, idx_v, rr_v, m_v, rows_v,
                   out_v, sem):
        wid = lax.axis_index("s") * NC + lax.axis_index("c")
        pltpu.sync_copy(idx_hbm.at[wid], idx_v)
        for j in range(n_chunks):
            for k in range(CHUNK // 16):
                r = idx_v[j, pl.ds(k * 16, 16)]
                rr = ((r >> 15) << 15) + ((r & 16383) << 1) + ((r >> 14) & 1)
                rr_v[j, pl.ds(k * 16, 16)] = rr
                m_v[j, pl.ds(k * 16, 16)] = ((rr >> 2) << 1) + (rr & 1)
        copies = [
            pltpu.async_copy(
                table_hbm.at[m_v.at[j]],
                rows_v.at[pl.ds(j * CHUNK, CHUNK)],
                sem,
            )
            for j in range(n_chunks)
        ]
        for cp in copies:
            cp.wait()

        def unpack_group(g, carry):
            labs = g * 16 + lax.iota(jnp.int32, 16)
            rrg = rr_v[g >> 3, pl.ds((g & 7) * 16, 16)]
            shv = (jnp.int32(1) - ((rrg >> 1) & 1)) << 4
            for q in range(D):
                qv = jnp.full((16,), q, jnp.int32)
                v = plsc.bitcast(plsc.load_gather(rows_v, [labs, qv]),
                                 jnp.int32)
                bits = (v << shv) & jnp.int32(-65536)
                plsc.store_scatter(out_v, [labs, qv],
                                   plsc.bitcast(bits, jnp.float32))
            return carry

        lax.fori_loop(0, b_per_w // 16, unpack_group, 0)
        pltpu.sync_copy(out_v, out_hbm.at[pl.ds(wid * b_per_w, b_per_w)])

    return emb_gather(idx, z64)


# speedup vs baseline: 3.1396x; 1.0002x over previous
import functools

import jax
import jax.numpy as jnp
from jax import lax
from jax.experimental import pallas as pl
from jax.experimental.pallas import tpu as pltpu
from jax.experimental.pallas import tpu_sc as plsc

NC = 2
NS = 16
NW = NC * NS
CHUNK = 128
LBLK = 32768
HALF = LBLK // 2


def _reformat_body(q0_ref, q1_ref, q2_ref, q3_ref, z_ref):
    qs = (q0_ref, q1_ref, q2_ref, q3_ref)
    s = jnp.concatenate(
        [q[:, 0:HALF] for q in qs] + [q[:, HALF:LBLK] for q in qs], axis=0
    )
    zb = s.T.astype(jnp.bfloat16)
    z_ref[...] = pltpu.bitcast(zb, jnp.float32)


def _tc_reformat(tbl_t):
    D, V = tbl_t.shape
    grid = (V + LBLK - 1) // LBLK
    return pl.pallas_call(
        _reformat_body,
        grid=(grid,),
        in_specs=[
            pl.BlockSpec((D // 4, LBLK), lambda j, i=i: (i, j))
            for i in range(4)
        ],
        out_specs=pl.BlockSpec((HALF // 2, 128), lambda j: (j, 0)),
        out_shape=jax.ShapeDtypeStruct((grid * (HALF // 2), 128), jnp.float32),
    )(tbl_t, tbl_t, tbl_t, tbl_t)


def kernel(condition, learned_embs, uncond_embedding):
    B = condition.shape[0]
    V, D = learned_embs.shape
    b_per_w = B // NW
    n_chunks = b_per_w // CHUNK

    idx = condition.astype(jnp.int32).reshape(NW, n_chunks, CHUNK)
    z = _tc_reformat(learned_embs.T)
    z64 = z.reshape(z.shape[0] * 2, D)

    mesh = plsc.VectorSubcoreMesh(core_axis_name="c", subcore_axis_name="s")

    @functools.partial(
        pl.kernel,
        mesh=mesh,
        out_type=jax.ShapeDtypeStruct((B, D), jnp.float32),
        scratch_types=[
            pltpu.VMEM((n_chunks, CHUNK), jnp.int32),
            pltpu.VMEM((n_chunks, CHUNK), jnp.int32),
            pltpu.VMEM((n_chunks, CHUNK), jnp.int32),
            pltpu.VMEM((b_per_w, D), jnp.float32),
            pltpu.VMEM((b_per_w, D), jnp.float32),
            pltpu.SemaphoreType.DMA,
        ],
        compiler_params=pltpu.CompilerParams(
            use_tc_tiling_on_sc=False, needs_layout_passes=False
        ),
    )
    def emb_gather(idx_hbm, table_hbm, out_hbm, idx_v, rr_v, m_v, rows_v,
                   out_v, sem):
        wid = lax.axis_index("s") * NC + lax.axis_index("c")
        pltpu.sync_copy(idx_hbm.at[wid], idx_v)
        for j in range(n_chunks):
            for k in range(CHUNK // 16):
                r = idx_v[j, pl.ds(k * 16, 16)]
                rr = ((r >> 15) << 15) + ((r & 16383) << 1) + ((r >> 14) & 1)
                rr_v[j, pl.ds(k * 16, 16)] = rr
                m_v[j, pl.ds(k * 16, 16)] = ((rr >> 2) << 1) + (rr & 1)
        copies = [
            pltpu.async_copy(
                table_hbm.at[m_v.at[j]],
                rows_v.at[pl.ds(j * CHUNK, CHUNK)],
                sem,
            )
            for j in range(n_chunks)
        ]
        for cp in copies:
            cp.wait()

        def unpack_group(g, carry):
            labs = g * 16 + lax.iota(jnp.int32, 16)
            rrg = rr_v[g >> 3, pl.ds((g & 7) * 16, 16)]
            shv = (jnp.int32(1) - ((rrg >> 1) & 1)) << 4
            for q in range(D):
                qv = jnp.full((16,), q, jnp.int32)
                v = plsc.bitcast(plsc.load_gather(rows_v, [labs, qv]),
                                 jnp.int32)
                bits = (v << shv) & jnp.int32(-65536)
                plsc.store_scatter(out_v, [labs, qv],
                                   plsc.bitcast(bits, jnp.float32))
            return carry

        lax.fori_loop(0, b_per_w // 16, unpack_group, 0)
        pltpu.sync_copy(out_v, out_hbm.at[pl.ds(wid * b_per_w, b_per_w)])

    return emb_gather(idx, z64)


# speedup vs baseline: 3.7163x; 1.1837x over previous
import functools

import jax
import jax.numpy as jnp
from jax import lax
from jax.experimental import pallas as pl
from jax.experimental.pallas import tpu as pltpu
from jax.experimental.pallas import tpu_sc as plsc

NC = 2
NS = 16
NW = NC * NS
CHUNK = 128
LBLK = 32768
HALF = LBLK // 2


def _reformat_body(q0_ref, q1_ref, q2_ref, q3_ref, z_ref):
    qs = (q0_ref, q1_ref, q2_ref, q3_ref)
    s = jnp.concatenate(
        [q[:, 0:HALF] for q in qs] + [q[:, HALF:LBLK] for q in qs], axis=0
    )
    zb = s.T.astype(jnp.bfloat16)
    z_ref[...] = pltpu.bitcast(zb, jnp.float32)


def _tc_reformat(tbl_t):
    D, V = tbl_t.shape
    grid = (V + LBLK - 1) // LBLK
    return pl.pallas_call(
        _reformat_body,
        grid=(grid,),
        in_specs=[
            pl.BlockSpec((D // 4, LBLK), lambda j, i=i: (i, j))
            for i in range(4)
        ],
        out_specs=pl.BlockSpec((HALF // 2, 128), lambda j: (j, 0)),
        out_shape=jax.ShapeDtypeStruct((grid * (HALF // 2), 128), jnp.float32),
    )(tbl_t, tbl_t, tbl_t, tbl_t)


def kernel(condition, learned_embs, uncond_embedding):
    B = condition.shape[0]
    V, D = learned_embs.shape
    b_per_w = B // NW
    n_chunks = b_per_w // CHUNK

    idx = condition.astype(jnp.int32).reshape(NW, n_chunks, CHUNK)
    z = _tc_reformat(learned_embs.T)
    z64 = z.reshape(z.shape[0] * 2, D)

    mesh = plsc.VectorSubcoreMesh(core_axis_name="c", subcore_axis_name="s")

    @functools.partial(
        pl.kernel,
        mesh=mesh,
        out_type=jax.ShapeDtypeStruct((B, D), jnp.float32),
        scratch_types=[
            pltpu.VMEM((n_chunks, CHUNK), jnp.int32),
            pltpu.VMEM((n_chunks, CHUNK), jnp.int32),
            pltpu.VMEM((n_chunks, CHUNK), jnp.int32),
            pltpu.VMEM((b_per_w, D), jnp.float32),
            pltpu.VMEM((b_per_w, D), jnp.float32),
            pltpu.SemaphoreType.DMA,
        ],
        compiler_params=pltpu.CompilerParams(
            use_tc_tiling_on_sc=False, needs_layout_passes=False
        ),
    )
    def emb_gather(idx_hbm, table_hbm, out_hbm, idx_v, rr_v, m_v, rows_v,
                   out_v, sem):
        wid = lax.axis_index("s") * NC + lax.axis_index("c")
        pltpu.sync_copy(idx_hbm.at[wid], idx_v)
        for j in range(n_chunks):
            for k in range(CHUNK // 16):
                r = idx_v[j, pl.ds(k * 16, 16)]
                rr = ((r >> 15) << 15) + ((r & 16383) << 1) + ((r >> 14) & 1)
                rr_v[j, pl.ds(k * 16, 16)] = rr
                m_v[j, pl.ds(k * 16, 16)] = ((rr >> 2) << 1) + (rr & 1)
        copies = [
            pltpu.async_copy(
                table_hbm.at[m_v.at[j]],
                rows_v.at[pl.ds(j * CHUNK, CHUNK)],
                sem,
            )
            for j in range(n_chunks)
        ]
        for cp in copies:
            cp.wait()

        def unpack_group(g, carry):
            rrg = rr_v[g >> 3, pl.ds((g & 7) * 16, 16)]
            shv = (jnp.int32(1) - ((rrg >> 1) & 1)) << 4
            mask = jnp.full((16,), -65536, jnp.int32)
            for l in range(16):
                b = g * 16 + l
                sh = lax.gather(
                    shv, jnp.full((16, 1), l, jnp.int32),
                    lax.GatherDimensionNumbers(
                        offset_dims=(), collapsed_slice_dims=(0,),
                        start_index_map=(0,)),
                    (1,), mode=lax.GatherScatterMode.PROMISE_IN_BOUNDS)
                for k in range(D // 16):
                    v = plsc.bitcast(rows_v[b, pl.ds(k * 16, 16)], jnp.int32)
                    out_v[b, pl.ds(k * 16, 16)] = plsc.bitcast(
                        (v << sh) & mask, jnp.float32)
            return carry

        lax.fori_loop(0, b_per_w // 16, unpack_group, 0)
        pltpu.sync_copy(out_v, out_hbm.at[pl.ds(wid * b_per_w, b_per_w)])

    return emb_gather(idx, z64)
